# baseline (speedup 1.0000x reference)
import functools
import jax
import jax.numpy as jnp
from jax import lax
from jax.experimental import pallas as pl
from jax.experimental.pallas import tpu as pltpu
from jax.experimental.pallas import tpu_sc as plsc

N = 100000
D = 128
B = 256
N_ITERS = 3
L = 16
NW = 32
SEG_PER_W = B // NW

_NEG_INF = float("-inf")


def _lstm_body(qs_ref, h_ref, c_ref, w_ih_ref, w_hh_ref, bias_ref,
               h_out, c_out):
    f32 = jnp.float32
    gates = (
        lax.dot_general(qs_ref[...], w_ih_ref[...], (((1,), (1,)), ((), ())),
                        precision=lax.Precision.HIGHEST,
                        preferred_element_type=f32)
        + lax.dot_general(h_ref[...], w_hh_ref[...], (((1,), (1,)), ((), ())),
                          precision=lax.Precision.HIGHEST,
                          preferred_element_type=f32)
        + bias_ref[...]
    )
    i_ = jax.nn.sigmoid(gates[:, 0 * D:1 * D])
    f_ = jax.nn.sigmoid(gates[:, 1 * D:2 * D])
    g_ = jnp.tanh(gates[:, 2 * D:3 * D])
    o_ = jax.nn.sigmoid(gates[:, 3 * D:4 * D])
    c_new = f_ * c_ref[...] + i_ * g_
    h_out[...] = o_ * jnp.tanh(c_new)
    c_out[...] = c_new


def _lstm_step(q_star, h, c, W_ih, W_hh, bias):
    return pl.pallas_call(
        _lstm_body,
        out_shape=(jax.ShapeDtypeStruct((B, D), jnp.float32),
                   jax.ShapeDtypeStruct((B, D), jnp.float32)),
    )(q_star, h, c, W_ih, W_hh, bias)


def _readout_body(feat_hbm, q_hbm, offs_hbm, out_hbm,
                  offs_v, q_v, stage_v, w_v, outst_v):
    f32 = jnp.float32
    i32 = jnp.int32
    wid = lax.axis_index("c") * 16 + lax.axis_index("s")
    pltpu.sync_copy(offs_hbm.at[pl.ds(wid * SEG_PER_W, 16)], offs_v)
    lanes = lax.iota(i32, L)
    off_vec = offs_v[...]

    for k in range(SEG_PER_W):
        b = wid * SEG_PER_W + k
        s0 = off_vec[k]
        s1 = off_vec[k + 1]
        pltpu.sync_copy(q_hbm.at[pl.ds(b * D, D)], q_v)
        base = (s0 // 8) * 8
        nch = jnp.where(s1 > s0, (s1 - base + (L - 1)) // L, 0)

        def chunk(j, carry, base=base, s0=s0, s1=s1):
            m, z, s_acc = carry
            sp = pl.multiple_of(jnp.minimum(base + j * L, N - L), 8)
            pltpu.sync_copy(feat_hbm.at[pl.ds(sp * D, L * D)], stage_v)
            ridx = sp + lanes
            valid = (ridx >= s0) & (ridx < s1)

            def dot_d(d, e):
                col = plsc.load_gather(stage_v, [lanes * D + d])
                qd = plsc.load_gather(q_v, [jnp.full((L,), d, i32)])
                return e + col * qd

            e = lax.fori_loop(0, D, dot_d, jnp.zeros((L,), f32), unroll=16)
            e = jnp.where(valid, e, _NEG_INF)
            cmax = jnp.broadcast_to(jnp.max(e), (L,))
            m_new = jnp.maximum(m, cmax)
            scale = jnp.exp(m - m_new)
            w = jnp.exp(e - m_new)
            z = z * scale + w
            w_v[...] = w

            def acc_r(r, s_acc):
                wr = plsc.load_gather(w_v, [jnp.full((L,), r, i32)])
                rbase = r * D + lanes
                return tuple(
                    s_acc[jj] + wr * plsc.load_gather(
                        stage_v, [rbase + jj * L])
                    for jj in range(D // L))

            s_new = tuple(sj * scale for sj in s_acc)
            s_new = lax.fori_loop(0, L, acc_r, s_new, unroll=4)
            return (m_new, z, s_new)

        init = (jnp.full((L,), _NEG_INF, f32), jnp.zeros((L,), f32),
                tuple(jnp.zeros((L,), f32) for _ in range(D // L)))
        m, z, s_acc = lax.fori_loop(0, nch, chunk, init)
        ztot = jnp.broadcast_to(jnp.sum(z), (L,))
        rcp_v = jnp.where(ztot > 0.0, 1.0 / ztot, 0.0)
        for jj in range(D // L):
            outst_v[pl.ds(k * D + jj * L, L)] = s_acc[jj] * rcp_v

    pltpu.sync_copy(outst_v,
                    out_hbm.at[pl.ds(wid * SEG_PER_W * D, SEG_PER_W * D)])


_sc_readout = functools.partial(
    pl.kernel,
    mesh=plsc.VectorSubcoreMesh(core_axis_name="c", subcore_axis_name="s"),
    compiler_params=pltpu.CompilerParams(needs_layout_passes=False),
    out_type=jax.ShapeDtypeStruct((B * D,), jnp.float32),
    scratch_types=[
        pltpu.VMEM((16,), jnp.int32),
        pltpu.VMEM((D,), jnp.float32),
        pltpu.VMEM((L * D,), jnp.float32),
        pltpu.VMEM((L,), jnp.float32),
        pltpu.VMEM((SEG_PER_W * D,), jnp.float32),
    ],
)(_readout_body)


@jax.jit
def kernel(feat, W_ih, W_hh, b_ih, b_hh, segment_ids):
    seg = segment_ids.astype(jnp.int32)
    offsets = jnp.searchsorted(seg, jnp.arange(B + 1, dtype=jnp.int32),
                               side="left").astype(jnp.int32)
    offs_pad = jnp.concatenate(
        [offsets, jnp.full((15,), N, jnp.int32)])
    bias = (b_ih + b_hh).reshape(1, 4 * D)
    feat_flat = feat.reshape(N * D)

    h = jnp.zeros((B, D), jnp.float32)
    c = jnp.zeros((B, D), jnp.float32)
    q_star = jnp.zeros((B, 2 * D), jnp.float32)
    for _ in range(N_ITERS):
        h, c = _lstm_step(q_star, h, c, W_ih, W_hh, bias)
        readout = _sc_readout(feat_flat, h.reshape(B * D), offs_pad)
        q_star = jnp.concatenate([h, readout.reshape(B, D)], axis=1)
    return q_star


# speedup vs baseline: 2.6730x; 2.6730x over previous
import functools
import jax
import jax.numpy as jnp
from jax import lax
from jax.experimental import pallas as pl
from jax.experimental.pallas import tpu as pltpu
from jax.experimental.pallas import tpu_sc as plsc

N = 100000
D = 128
B = 256
N_ITERS = 3
L = 16
NW = 32
SEG_PER_W = B // NW
SZ = 256

_NEG_INF = float("-inf")


def _lstm_body(qs_ref, h_ref, c_ref, w_ih_ref, w_hh_ref, bias_ref,
               h_out, c_out):
    f32 = jnp.float32
    gates = (
        lax.dot_general(qs_ref[...], w_ih_ref[...], (((1,), (1,)), ((), ())),
                        precision=lax.Precision.HIGHEST,
                        preferred_element_type=f32)
        + lax.dot_general(h_ref[...], w_hh_ref[...], (((1,), (1,)), ((), ())),
                          precision=lax.Precision.HIGHEST,
                          preferred_element_type=f32)
        + bias_ref[...]
    )
    i_ = jax.nn.sigmoid(gates[:, 0 * D:1 * D])
    f_ = jax.nn.sigmoid(gates[:, 1 * D:2 * D])
    g_ = jnp.tanh(gates[:, 2 * D:3 * D])
    o_ = jax.nn.sigmoid(gates[:, 3 * D:4 * D])
    c_new = f_ * c_ref[...] + i_ * g_
    h_out[...] = o_ * jnp.tanh(c_new)
    c_out[...] = c_new


def _lstm_step(q_star, h, c, W_ih, W_hh, bias):
    return pl.pallas_call(
        _lstm_body,
        out_shape=(jax.ShapeDtypeStruct((B, D), jnp.float32),
                   jax.ShapeDtypeStruct((B, D), jnp.float32)),
    )(q_star, h, c, W_ih, W_hh, bias)


def _readout_body(feat_hbm, q_hbm, offs_hbm, out_hbm,
                  offs_v, q_v, stage_v, ptile_v, w_v, outst_v, dma_sem):
    f32 = jnp.float32
    i32 = jnp.int32
    wid = lax.axis_index("c") * 16 + lax.axis_index("s")
    pltpu.sync_copy(offs_hbm.at[pl.ds(wid * SEG_PER_W, 16)], offs_v)
    lanes = lax.iota(i32, L)
    off_vec = offs_v[...]

    def stage_src(start):
        sp = pl.multiple_of(jnp.minimum(start, N - SZ), 8)
        return sp, feat_hbm.at[pl.ds(sp * D, SZ * D)]

    def stage_dst(slot):
        off = pl.multiple_of(slot * (SZ * D), 8)
        return stage_v.at[pl.ds(off, SZ * D)]

    for k in range(SEG_PER_W):
        b = wid * SEG_PER_W + k
        s0 = off_vec[k]
        s1 = off_vec[k + 1]
        pltpu.sync_copy(q_hbm.at[pl.ds(b * D, D)], q_v)
        q8 = [q_v[pl.ds(jj * L, L)] for jj in range(D // L)]
        base = (s0 // 8) * 8
        nst = jnp.where(s1 > s0, (s1 - base + (SZ - 1)) // SZ, 0)

        @pl.when(nst > 0)
        def _():
            _, src = stage_src(base)
            pltpu.async_copy(src, stage_dst(0), dma_sem)

        def stage_loop(st, carry, base=base, s0=s0, s1=s1, q8=q8):
            m, z, s_acc = carry
            start = base + st * SZ
            sp, _ = stage_src(start)
            slot = lax.rem(st, 2)
            @pl.when(st + 1 < jnp.where(s1 > s0,
                                        (s1 - base + (SZ - 1)) // SZ, 0))
            def _():
                _, nsrc = stage_src(start + SZ)
                pltpu.async_copy(nsrc, stage_dst(1 - slot), dma_sem)

            _, dummy_src = stage_src(base)
            pltpu.make_async_copy(dummy_src, stage_dst(slot), dma_sem).wait()

            lo = jnp.maximum(s0, start)
            hi = jnp.minimum(s1, start + SZ)
            nsub = (hi - sp + (L - 1)) // L
            slot_off = slot * (SZ * D)

            def chunk(c, carry2, sp=sp, lo=lo, hi=hi, slot_off=slot_off,
                      q8=q8):
                m, z, s_acc = carry2
                cb = sp + c * L
                coff = slot_off + c * (L * D)
                ridx = cb + lanes
                valid = (ridx >= lo) & (ridx < hi)

                for r in range(L):
                    roff = coff + r * D
                    p = jnp.zeros((L,), f32)
                    for jj in range(D // L):
                        blk = plsc.load_gather(
                            stage_v, [roff + jj * L + lanes])
                        p = p + blk * q8[jj]
                    ptile_v[pl.ds(r * L, L)] = p

                e = jnp.zeros((L,), f32)
                for cc in range(L):
                    e = e + plsc.load_gather(ptile_v, [lanes * L + cc])

                e = jnp.where(valid, e, _NEG_INF)
                cmax = jnp.broadcast_to(jnp.max(e), (L,))
                m_new = jnp.maximum(m, cmax)
                scale = jnp.where(m == m_new, 1.0, jnp.exp(m - m_new))
                w = jnp.where(valid, jnp.exp(e - m_new), 0.0)
                z = z * scale + w
                w_v[...] = w

                def acc_r(r, s_acc, coff=coff):
                    wr = plsc.load_gather(w_v, [jnp.full((L,), r, i32)])
                    rbase = coff + r * D + lanes
                    return tuple(
                        s_acc[jj] + wr * plsc.load_gather(
                            stage_v, [rbase + jj * L])
                        for jj in range(D // L))

                s_new = tuple(sj * scale for sj in s_acc)
                s_new = lax.fori_loop(0, L, acc_r, s_new, unroll=4)
                return (m_new, z, s_new)

            return lax.fori_loop(0, nsub, chunk, (m, z, s_acc))

        init = (jnp.full((L,), _NEG_INF, f32), jnp.zeros((L,), f32),
                tuple(jnp.zeros((L,), f32) for _ in range(D // L)))
        m, z, s_acc = lax.fori_loop(0, nst, stage_loop, init)
        ztot = jnp.broadcast_to(jnp.sum(z), (L,))
        rcp_v = jnp.where(ztot > 0.0, 1.0 / ztot, 0.0)
        for jj in range(D // L):
            outst_v[pl.ds(k * D + jj * L, L)] = s_acc[jj] * rcp_v

    pltpu.sync_copy(outst_v,
                    out_hbm.at[pl.ds(wid * SEG_PER_W * D, SEG_PER_W * D)])


_sc_readout = functools.partial(
    pl.kernel,
    mesh=plsc.VectorSubcoreMesh(core_axis_name="c", subcore_axis_name="s"),
    compiler_params=pltpu.CompilerParams(needs_layout_passes=False),
    out_type=jax.ShapeDtypeStruct((B * D,), jnp.float32),
    scratch_types=[
        pltpu.VMEM((16,), jnp.int32),
        pltpu.VMEM((D,), jnp.float32),
        pltpu.VMEM((2 * SZ * D,), jnp.float32),
        pltpu.VMEM((L * L,), jnp.float32),
        pltpu.VMEM((L,), jnp.float32),
        pltpu.VMEM((SEG_PER_W * D,), jnp.float32),
        pltpu.SemaphoreType.DMA,
    ],
)(_readout_body)


@jax.jit
def kernel(feat, W_ih, W_hh, b_ih, b_hh, segment_ids):
    seg = segment_ids.astype(jnp.int32)
    offsets = jnp.searchsorted(seg, jnp.arange(B + 1, dtype=jnp.int32),
                               side="left").astype(jnp.int32)
    offs_pad = jnp.concatenate(
        [offsets, jnp.full((15,), N, jnp.int32)])
    bias = (b_ih + b_hh).reshape(1, 4 * D)
    feat_flat = feat.reshape(N * D)

    h = jnp.zeros((B, D), jnp.float32)
    c = jnp.zeros((B, D), jnp.float32)
    q_star = jnp.zeros((B, 2 * D), jnp.float32)
    for _ in range(N_ITERS):
        h, c = _lstm_step(q_star, h, c, W_ih, W_hh, bias)
        readout = _sc_readout(feat_flat, h.reshape(B * D), offs_pad)
        q_star = jnp.concatenate([h, readout.reshape(B, D)], axis=1)
    return q_star


# speedup vs baseline: 2.7447x; 1.0268x over previous
import functools
import jax
import jax.numpy as jnp
from jax import lax
from jax.experimental import pallas as pl
from jax.experimental.pallas import tpu as pltpu
from jax.experimental.pallas import tpu_sc as plsc

N = 100000
D = 128
B = 256
N_ITERS = 3
L = 16
NW = 32
SEG_PER_W = B // NW
SZ = 256
PT = L + 1

_NEG_INF = float("-inf")


def _lstm_body(q_ref, r_ref, h_ref, c_ref, w_q_ref, w_r_ref, w_hh_ref,
               bias_ref, h_out, c_out):
    f32 = jnp.float32
    dims = (((1,), (1,)), ((), ()))
    hp = lax.Precision.HIGHEST
    gates = (
        lax.dot_general(q_ref[...], w_q_ref[...], dims, precision=hp,
                        preferred_element_type=f32)
        + lax.dot_general(r_ref[...], w_r_ref[...], dims, precision=hp,
                          preferred_element_type=f32)
        + lax.dot_general(h_ref[...], w_hh_ref[...], dims, precision=hp,
                          preferred_element_type=f32)
        + bias_ref[...]
    )
    i_ = jax.nn.sigmoid(gates[:, 0 * D:1 * D])
    f_ = jax.nn.sigmoid(gates[:, 1 * D:2 * D])
    g_ = jnp.tanh(gates[:, 2 * D:3 * D])
    o_ = jax.nn.sigmoid(gates[:, 3 * D:4 * D])
    c_new = f_ * c_ref[...] + i_ * g_
    h_out[...] = o_ * jnp.tanh(c_new)
    c_out[...] = c_new


def _lstm_step(q, r, h, c, W_q, W_r, W_hh, bias):
    return pl.pallas_call(
        _lstm_body,
        out_shape=(jax.ShapeDtypeStruct((B, D), jnp.float32),
                   jax.ShapeDtypeStruct((B, D), jnp.float32)),
    )(q, r, h, c, W_q, W_r, W_hh, bias)


def _readout_body(feat_hbm, q_hbm, offs_hbm, out_hbm,
                  offs_v, q_v, stage_v, ptile_v, w_v, outst_v, dma_sem):
    f32 = jnp.float32
    i32 = jnp.int32
    wid = lax.axis_index("c") * 16 + lax.axis_index("s")
    pltpu.sync_copy(offs_hbm.at[pl.ds(wid * SEG_PER_W, 16)], offs_v)
    lanes = lax.iota(i32, L)
    off_vec = offs_v[...]

    def stage_src(start):
        sp = pl.multiple_of(jnp.minimum(start, N - SZ), 8)
        return sp, feat_hbm.at[pl.ds(sp * D, SZ * D)]

    def stage_dst(slot):
        off = pl.multiple_of(slot * (SZ * D), 8)
        return stage_v.at[pl.ds(off, SZ * D)]

    for k in range(SEG_PER_W):
        b = wid * SEG_PER_W + k
        s0 = off_vec[k]
        s1 = off_vec[k + 1]
        pltpu.sync_copy(q_hbm.at[pl.ds(b * D, D)], q_v)
        q8 = [q_v[pl.ds(jj * L, L)] for jj in range(D // L)]
        base = (s0 // 8) * 8
        nst = jnp.where(s1 > s0, (s1 - base + (SZ - 1)) // SZ, 0)

        @pl.when(nst > 0)
        def _():
            _, src = stage_src(base)
            pltpu.async_copy(src, stage_dst(0), dma_sem)

        def stage_loop(st, carry, base=base, s0=s0, s1=s1, q8=q8, nst=nst):
            m, z, s_acc = carry
            start = base + st * SZ
            sp, _ = stage_src(start)
            slot = lax.rem(st, 2)
            @pl.when(st + 1 < nst)
            def _():
                _, nsrc = stage_src(start + SZ)
                pltpu.async_copy(nsrc, stage_dst(1 - slot), dma_sem)

            _, dummy_src = stage_src(base)
            pltpu.make_async_copy(dummy_src, stage_dst(slot), dma_sem).wait()

            lo = jnp.maximum(s0, start)
            hi = jnp.minimum(s1, start + SZ)
            nsub = (hi - sp + (L - 1)) // L
            slot_off = slot * (SZ * D)

            def chunk(c, carry2, sp=sp, lo=lo, hi=hi, slot_off=slot_off,
                      q8=q8):
                m, z, s_acc = carry2
                cb = sp + c * L
                coff = slot_off + c * (L * D)
                ridx = cb + lanes
                valid = (ridx >= lo) & (ridx < hi)

                for r in range(L):
                    roff = coff + r * D
                    p = jnp.zeros((L,), f32)
                    for jj in range(D // L):
                        p = p + stage_v[pl.ds(roff + jj * L, L)] * q8[jj]
                    plsc.store_scatter(ptile_v, [lanes + r * PT], p)

                e = jnp.zeros((L,), f32)
                for cc in range(L):
                    e = e + plsc.load_gather(ptile_v, [lanes * PT + cc])

                e = jnp.where(valid, e, _NEG_INF)
                cmax = jnp.broadcast_to(jnp.max(e), (L,))
                m_new = jnp.maximum(m, cmax)
                scale = jnp.where(m == m_new, 1.0, jnp.exp(m - m_new))
                w = jnp.where(valid, jnp.exp(e - m_new), 0.0)
                z = z * scale + w
                w_v[...] = w

                s_new = list(sj * scale for sj in s_acc)
                for r in range(L):
                    wr = plsc.load_gather(w_v, [jnp.full((L,), r, i32)])
                    rbase = coff + r * D
                    for jj in range(D // L):
                        s_new[jj] = (s_new[jj]
                                     + wr * stage_v[pl.ds(rbase + jj * L, L)])
                return (m_new, z, tuple(s_new))

            return lax.fori_loop(0, nsub, chunk, (m, z, s_acc))

        init = (jnp.full((L,), _NEG_INF, f32), jnp.zeros((L,), f32),
                tuple(jnp.zeros((L,), f32) for _ in range(D // L)))
        m, z, s_acc = lax.fori_loop(0, nst, stage_loop, init)
        ztot = jnp.broadcast_to(jnp.sum(z), (L,))
        rcp_v = jnp.where(ztot > 0.0, 1.0 / ztot, 0.0)
        for jj in range(D // L):
            outst_v[pl.ds(k * D + jj * L, L)] = s_acc[jj] * rcp_v

    pltpu.sync_copy(outst_v,
                    out_hbm.at[pl.ds(wid * SEG_PER_W * D, SEG_PER_W * D)])


_sc_readout = functools.partial(
    pl.kernel,
    mesh=plsc.VectorSubcoreMesh(core_axis_name="c", subcore_axis_name="s"),
    compiler_params=pltpu.CompilerParams(needs_layout_passes=False),
    out_type=jax.ShapeDtypeStruct((B * D,), jnp.float32),
    scratch_types=[
        pltpu.VMEM((16,), jnp.int32),
        pltpu.VMEM((D,), jnp.float32),
        pltpu.VMEM((2 * SZ * D,), jnp.float32),
        pltpu.VMEM((L * PT,), jnp.float32),
        pltpu.VMEM((L,), jnp.float32),
        pltpu.VMEM((SEG_PER_W * D,), jnp.float32),
        pltpu.SemaphoreType.DMA,
    ],
)(_readout_body)


@jax.jit
def kernel(feat, W_ih, W_hh, b_ih, b_hh, segment_ids):
    seg = segment_ids.astype(jnp.int32)
    offsets = jnp.searchsorted(seg, jnp.arange(B + 1, dtype=jnp.int32),
                               side="left").astype(jnp.int32)
    offs_pad = jnp.concatenate(
        [offsets, jnp.full((15,), N, jnp.int32)])
    bias = (b_ih + b_hh).reshape(1, 4 * D)
    W_q = W_ih[:, :D]
    W_r = W_ih[:, D:]
    feat_flat = feat.reshape(N * D)

    h = jnp.zeros((B, D), jnp.float32)
    c = jnp.zeros((B, D), jnp.float32)
    q = jnp.zeros((B, D), jnp.float32)
    readout = jnp.zeros((B, D), jnp.float32)
    for _ in range(N_ITERS):
        h, c = _lstm_step(q, readout, h, c, W_q, W_r, W_hh, bias)
        q = h
        readout = _sc_readout(feat_flat, q.reshape(B * D), offs_pad)
        readout = readout.reshape(B, D)
    return jnp.concatenate([q, readout], axis=1)
